# baseline (speedup 1.0000x reference)
import functools

import jax
import jax.numpy as jnp
from jax import lax
from jax.experimental import pallas as pl
from jax.experimental.pallas import tpu as pltpu
from jax.experimental.pallas import tpu_sc as plsc

_NC = 2
_NS = 16
_NW = _NC * _NS

_E_PAD = 327680
_GATHER_CHUNK = 512
_SCAT_CHUNK = 1024
_AROW = 5
_ACOL = 2048
_NPAD = _AROW * _ACOL
_EDGE_BLK = 2560


def _tables_body(xin_ref, pos_ref, wx_ref, wp_ref, b_ref, t_ref):
    pb = jax.lax.dot_general(pos_ref[...], wp_ref[...], (((1,), (0,)), ((), ())),
                             preferred_element_type=jnp.float32)
    xa = jax.lax.dot_general(xin_ref[...], wx_ref[...], (((1,), (0,)), ((), ())),
                             preferred_element_type=jnp.float32)
    t_ref[...] = xa + pb + b_ref[...]


def _make_tables(xin, pos, wx, wp, b):
    n, h = xin.shape[0], wx.shape[1]
    return pl.pallas_call(
        _tables_body,
        out_shape=jax.ShapeDtypeStruct((n, h), jnp.float32),
    )(xin, pos, wx, wp, b.reshape(1, h))


def _make_mlp_body(n_valid_blk):
    def _mlp_body(ga_ref, gp_ref, wp_ref, w2_ref, b2_ref, w3_ref, b3_ref,
                  out_ref):
        i = pl.program_id(0)

        @pl.when(i < n_valid_blk)
        def _():
            pb = jnp.dot(gp_ref[:, :3], wp_ref[...],
                         preferred_element_type=jnp.float32)
            g = jnp.maximum(ga_ref[...] - pb, 0.0)
            h = jnp.maximum(
                jnp.dot(g, w2_ref[...], preferred_element_type=jnp.float32)
                + b2_ref[...], 0.0)
            m = jnp.maximum(
                jnp.dot(h, w3_ref[...], preferred_element_type=jnp.float32)
                + b3_ref[...], 0.0)
            out_ref[...] = m.T

        @pl.when(i >= n_valid_blk)
        def _():
            out_ref[...] = jnp.zeros_like(out_ref)

    return _mlp_body


def _edge_mlp(ga, gp, wp, w2, b2, w3, b3, n_valid_blk):
    e = ga.shape[0]
    f = w3.shape[1]
    nblk = e // _EDGE_BLK
    return pl.pallas_call(
        _make_mlp_body(n_valid_blk),
        grid=(nblk,),
        in_specs=[
            pl.BlockSpec((_EDGE_BLK, 64), lambda i: (i, 0)),
            pl.BlockSpec((_EDGE_BLK, 16), lambda i: (i, 0)),
            pl.BlockSpec((3, 64), lambda i: (0, 0)),
            pl.BlockSpec((64, w2.shape[1]), lambda i: (0, 0)),
            pl.BlockSpec((1, w2.shape[1]), lambda i: (0, 0)),
            pl.BlockSpec((w2.shape[1], f), lambda i: (0, 0)),
            pl.BlockSpec((1, f), lambda i: (0, 0)),
        ],
        out_specs=pl.BlockSpec((f, _EDGE_BLK), lambda i: (0, i)),
        out_shape=jax.ShapeDtypeStruct((f, e), jnp.float32),
    )(ga, gp, wp, w2, b2.reshape(1, -1), w3, b3.reshape(1, -1))


def _post_body(aggp_ref, pos_ref, wg_ref, bg_ref, wx_ref, wp_ref, b1_ref,
               t_ref):
    aggt = jnp.max(aggp_ref[...], axis=0)[:, :pos_ref.shape[0]]
    h = jnp.maximum(
        jax.lax.dot_general(aggt, wg_ref[...], (((0,), (0,)), ((), ())),
                            preferred_element_type=jnp.float32) + bg_ref[...],
        0.0)
    pb = jax.lax.dot_general(pos_ref[...], wp_ref[...], (((1,), (0,)), ((), ())),
                             preferred_element_type=jnp.float32)
    xa = jax.lax.dot_general(h, wx_ref[...], (((1,), (0,)), ((), ())),
                             preferred_element_type=jnp.float32)
    t_ref[...] = xa + pb + b1_ref[...]


def _post_layer1(aggp, pos, wg, bg, wx, wp, b1):
    n = pos.shape[0]
    h2 = wx.shape[1]
    return pl.pallas_call(
        _post_body,
        out_shape=jax.ShapeDtypeStruct((n, h2), jnp.float32),
    )(aggp, pos, wg, bg.reshape(1, -1), wx, wp, b1.reshape(1, -1))


def _head_body(aggp_ref, wg_ref, bg_ref, w1_ref, b1_ref, w2_ref, b2_ref,
               out_ref):
    aggt = jnp.max(aggp_ref[...], axis=0)[:, :out_ref.shape[0]]
    h = jnp.maximum(
        jax.lax.dot_general(aggt, wg_ref[...], (((0,), (0,)), ((), ())),
                            preferred_element_type=jnp.float32) + bg_ref[...],
        0.0)
    z = jnp.maximum(
        jnp.dot(h, w1_ref[...], preferred_element_type=jnp.float32) + b1_ref[...],
        0.0)
    out_ref[...] = (
        jnp.dot(z, w2_ref[...], preferred_element_type=jnp.float32) + b2_ref[...])


def _head(aggp, wg, bg, w1, b1, w2, b2, n):
    return pl.pallas_call(
        _head_body,
        out_shape=jax.ShapeDtypeStruct((n, w2.shape[1]), jnp.float32),
    )(aggp, wg, bg.reshape(1, -1), w1, b1.reshape(1, -1), w2, b2.reshape(1, -1))


def _sc_gather_ap(a_tab, p_tab, src, dst):
    e = src.shape[0]
    epw = e // _NW
    ch = _GATHER_CHUNK
    mesh = plsc.VectorSubcoreMesh(core_axis_name="c", subcore_axis_name="s")

    @functools.partial(
        pl.kernel,
        out_type=(jax.ShapeDtypeStruct((e, 64), jnp.float32),
                  jax.ShapeDtypeStruct((e, 16), jnp.float32)),
        mesh=mesh,
        scratch_types=[
            pltpu.VMEM((ch,), jnp.int32),
            pltpu.VMEM((ch,), jnp.int32),
            pltpu.VMEM((ch, 64), jnp.float32),
            pltpu.VMEM((ch, 16), jnp.float32),
            pltpu.VMEM((ch,), jnp.int32),
            pltpu.VMEM((ch,), jnp.int32),
            pltpu.VMEM((ch, 64), jnp.float32),
            pltpu.VMEM((ch, 16), jnp.float32),
            pltpu.SemaphoreType.DMA,
            pltpu.SemaphoreType.DMA,
            pltpu.SemaphoreType.DMA,
            pltpu.SemaphoreType.DMA,
        ],
        compiler_params=pltpu.CompilerParams(use_tc_tiling_on_sc=False),
    )
    def k(a_hbm, p_hbm, src_hbm, dst_hbm, ga_hbm, gp_hbm, idxs0, idxd0,
          ts0, tp0, idxs1, idxd1, ts1, tp1, sa0, sb0, sa1, sb1):
        wid = lax.axis_index("s") * _NC + lax.axis_index("c")
        base = wid * epw

        @pl.loop(0, epw, step=2 * ch)
        def _(off):
            pltpu.sync_copy(src_hbm.at[pl.ds(base + off, ch)], idxs0)
            pltpu.sync_copy(dst_hbm.at[pl.ds(base + off, ch)], idxd0)
            ca0 = pltpu.async_copy(a_hbm.at[idxs0], ts0, sa0)
            cb0 = pltpu.async_copy(p_hbm.at[idxd0], tp0, sb0)
            pltpu.sync_copy(src_hbm.at[pl.ds(base + off + ch, ch)], idxs1)
            pltpu.sync_copy(dst_hbm.at[pl.ds(base + off + ch, ch)], idxd1)
            ca1 = pltpu.async_copy(a_hbm.at[idxs1], ts1, sa1)
            cb1 = pltpu.async_copy(p_hbm.at[idxd1], tp1, sb1)
            ca0.wait()
            cb0.wait()
            pltpu.sync_copy(ts0, ga_hbm.at[pl.ds(base + off, ch)])
            pltpu.sync_copy(tp0, gp_hbm.at[pl.ds(base + off, ch)])
            ca1.wait()
            cb1.wait()
            pltpu.sync_copy(ts1, ga_hbm.at[pl.ds(base + off + ch, ch)])
            pltpu.sync_copy(tp1, gp_hbm.at[pl.ds(base + off + ch, ch)])

    return k(a_tab, p_tab, src, dst)


def _sc_gather_a(a_tab, src):
    e = src.shape[0]
    epw = e // _NW
    ch = _GATHER_CHUNK
    mesh = plsc.VectorSubcoreMesh(core_axis_name="c", subcore_axis_name="s")

    @functools.partial(
        pl.kernel,
        out_type=jax.ShapeDtypeStruct((e, 64), jnp.float32),
        mesh=mesh,
        scratch_types=[
            pltpu.VMEM((ch,), jnp.int32),
            pltpu.VMEM((ch, 64), jnp.float32),
            pltpu.VMEM((ch,), jnp.int32),
            pltpu.VMEM((ch, 64), jnp.float32),
            pltpu.SemaphoreType.DMA,
            pltpu.SemaphoreType.DMA,
        ],
        compiler_params=pltpu.CompilerParams(use_tc_tiling_on_sc=False),
    )
    def k(a_hbm, src_hbm, ga_hbm, idxs0, ts0, idxs1, ts1, sa0, sa1):
        wid = lax.axis_index("s") * _NC + lax.axis_index("c")
        base = wid * epw

        @pl.loop(0, epw, step=2 * ch)
        def _(off):
            pltpu.sync_copy(src_hbm.at[pl.ds(base + off, ch)], idxs0)
            ca0 = pltpu.async_copy(a_hbm.at[idxs0], ts0, sa0)
            pltpu.sync_copy(src_hbm.at[pl.ds(base + off + ch, ch)], idxs1)
            ca1 = pltpu.async_copy(a_hbm.at[idxs1], ts1, sa1)
            ca0.wait()
            pltpu.sync_copy(ts0, ga_hbm.at[pl.ds(base + off, ch)])
            ca1.wait()
            pltpu.sync_copy(ts1, ga_hbm.at[pl.ds(base + off + ch, ch)])

    return k(a_tab, src)


_FPW = 4


def _sc_segment_max(msg_t, dst, n):
    f, e = msg_t.shape
    n_tr = f // _FPW
    n_seg = _NW // n_tr
    seg = e // n_seg
    ch = _SCAT_CHUNK
    mesh = plsc.VectorSubcoreMesh(core_axis_name="c", subcore_axis_name="s")

    @functools.partial(
        pl.kernel,
        out_type=jax.ShapeDtypeStruct((_NW * _FPW, _AROW, _ACOL),
                                      jnp.float32),
        mesh=mesh,
        scratch_types=[
            pltpu.VMEM((ch,), jnp.int32),
            pltpu.VMEM((_FPW, ch), jnp.float32),
        ] + [pltpu.VMEM((_AROW, _ACOL), jnp.float32)
             for _ in range(_FPW)] + [
            pltpu.VMEM((_AROW, _ACOL), jnp.int32),
            pltpu.VMEM((16,), jnp.int32),
            pltpu.SemaphoreType.DMA,
        ],
        compiler_params=pltpu.CompilerParams(needs_layout_passes=False),
    )
    def k(mt_hbm, dst_hbm, agg_hbm, idx_v, val_v, a0, a1, a2, a3, lanes,
          mask_v, sem):
        accs = (a0, a1, a2, a3)
        wid = lax.axis_index("s") * _NC + lax.axis_index("c")
        tr = wid % n_tr
        sg = wid // n_tr
        base = sg * seg
        zeros16 = jnp.zeros((16,), jnp.float32)
        lane_iota = lax.iota(jnp.int32, 16)

        for fr in range(_FPW):
            for r in range(_AROW):
                @pl.loop(0, _ACOL, step=16)
                def _(i):
                    accs[fr][r, pl.ds(i, 16)] = zeros16

        @pl.loop(0, seg, step=ch)
        def _(off):
            pltpu.sync_copy(dst_hbm.at[pl.ds(base + off, ch)], idx_v)
            pltpu.sync_copy(
                mt_hbm.at[pl.ds(tr * _FPW, _FPW), pl.ds(base + off, ch)],
                val_v)

            @pl.loop(0, ch, step=64)
            def _(i):
                idxs = [idx_v[pl.ds(i + 16 * k2, 16)] for k2 in range(4)]
                rs = [jnp.right_shift(ix, 11) for ix in idxs]
                cs = [jnp.bitwise_and(ix, _ACOL - 1) for ix in idxs]
                for k2 in range(4):
                    plsc.store_scatter(lanes, [rs[k2], cs[k2]],
                                       lane_iota + 16 * k2)
                rds = [plsc.load_gather(lanes, [rs[k2], cs[k2]])
                       for k2 in range(4)]
                neq = rds[0] != lane_iota
                for k2 in range(1, 4):
                    neq = jnp.logical_or(neq,
                                         rds[k2] != lane_iota + 16 * k2)
                dup64 = jnp.any(neq)

                @pl.when(jnp.logical_not(dup64))
                def _():
                    curs = [[plsc.load_gather(accs[fr], [rs[k2], cs[k2]])
                             for fr in range(_FPW)] for k2 in range(4)]
                    for k2 in range(4):
                        for fr in range(_FPW):
                            plsc.store_scatter(
                                accs[fr], [rs[k2], cs[k2]],
                                jnp.maximum(curs[k2][fr],
                                            val_v[fr, pl.ds(i + 16 * k2,
                                                            16)]))

                @pl.when(dup64)
                def _():
                    for k2 in range(4):
                        rr, cc = rs[k2], cs[k2]
                        plsc.store_scatter(lanes, [rr, cc], lane_iota)
                        rd = plsc.load_gather(lanes, [rr, cc])
                        vals = [val_v[fr, pl.ds(i + 16 * k2, 16)]
                                for fr in range(_FPW)]
                        has_dup = jnp.any(rd != lane_iota)

                        @pl.when(jnp.logical_not(has_dup))
                        def _():
                            curs2 = [plsc.load_gather(accs[fr], [rr, cc])
                                     for fr in range(_FPW)]
                            for fr in range(_FPW):
                                plsc.store_scatter(
                                    accs[fr], [rr, cc],
                                    jnp.maximum(curs2[fr], vals[fr]))

                        @pl.when(has_dup)
                        def _():
                            for fr in range(_FPW):
                                mask_v[...] = jnp.ones((16,), jnp.int32)

                                @pl.loop(0, 16)
                                def _(t):
                                    f2 = mask_v[...] > 0
                                    cur2 = plsc.load_gather(
                                        accs[fr], [rr, cc])
                                    new2 = jnp.maximum(cur2, vals[fr])
                                    plsc.store_scatter(accs[fr], [rr, cc],
                                                       new2, mask=f2)
                                    chk2 = plsc.load_gather(
                                        accs[fr], [rr, cc])
                                    mask_v[...] = jnp.where(
                                        f2 & (chk2 < new2), 1,
                                        0).astype(jnp.int32)

        for fr in range(_FPW):
            pltpu.async_copy(
                accs[fr],
                agg_hbm.at[(sg * n_tr + tr) * _FPW + fr],
                sem).wait()

    out = k(msg_t, dst)
    return out.reshape(n_seg, f, _NPAD)


def kernel(x, pos, edge_index, l1_W1, l1_b1, l1_W2, l1_b2, l1_W3, l1_b3,
           l1_Wg, l1_bg, l2_W1, l2_b1, l2_W2, l2_b2, l2_W3, l2_b3, l2_Wg,
           l2_bg, h_W1, h_b1, h_W2, h_b2):
    n = x.shape[0]
    e = edge_index.shape[1]
    src = edge_index[0].astype(jnp.int32)
    dst = edge_index[1].astype(jnp.int32)
    pos = pos.astype(jnp.float32)
    src_p = jnp.pad(src, (0, _E_PAD - e))
    dst_p = jnp.pad(dst, (0, _E_PAD - e))
    p16 = jnp.pad(pos, ((0, 0), (0, 13)))
    n_valid_blk = e // _EDGE_BLK

    a1 = _make_tables(x, pos, l1_W1[:128], l1_W1[128:], l1_b1)
    ga1, gp = _sc_gather_ap(a1, p16, src_p, dst_p)
    m1t = _edge_mlp(ga1, gp, l1_W1[128:], l1_W2, l1_b2, l1_W3, l1_b3,
                    n_valid_blk)
    agg1p = _sc_segment_max(m1t, dst_p, n)

    a2 = _post_layer1(agg1p, pos, l1_Wg, l1_bg, l2_W1[:32], l2_W1[32:], l2_b1)
    ga2 = _sc_gather_a(a2, src_p)
    m2t = _edge_mlp(ga2, gp, l2_W1[32:], l2_W2, l2_b2, l2_W3, l2_b3,
                    n_valid_blk)
    agg2p = _sc_segment_max(m2t, dst_p, n)

    return _head(agg2p, l2_Wg, l2_bg, h_W1, h_b1, h_W2, h_b2, n)


# speedup vs baseline: 1.0687x; 1.0687x over previous
import functools

import jax
import jax.numpy as jnp
from jax import lax
from jax.experimental import pallas as pl
from jax.experimental.pallas import tpu as pltpu
from jax.experimental.pallas import tpu_sc as plsc

_NC = 2
_NS = 16
_NW = _NC * _NS

_E_PAD = 327680
_GATHER_CHUNK = 512
_SCAT_CHUNK = 2048
_EDGE_BLK = 2560


def _tables_body(xin_ref, pos_ref, wx_ref, wp_ref, b_ref, t_ref):
    pb = jax.lax.dot_general(pos_ref[...], wp_ref[...], (((1,), (0,)), ((), ())),
                             preferred_element_type=jnp.float32)
    xa = jax.lax.dot_general(xin_ref[...], wx_ref[...], (((1,), (0,)), ((), ())),
                             preferred_element_type=jnp.float32)
    t_ref[...] = xa + pb + b_ref[...]


def _make_tables(xin, pos, wx, wp, b):
    n, h = xin.shape[0], wx.shape[1]
    return pl.pallas_call(
        _tables_body,
        out_shape=jax.ShapeDtypeStruct((n, h), jnp.float32),
    )(xin, pos, wx, wp, b.reshape(1, h))


def _make_mlp_body(n_valid_blk):
    def _mlp_body(ga_ref, gp_ref, wp_ref, w2_ref, b2_ref, w3_ref, b3_ref,
                  out_ref):
        i = pl.program_id(0)

        @pl.when(i < n_valid_blk)
        def _():
            pb = jnp.dot(gp_ref[:, :3], wp_ref[...],
                         preferred_element_type=jnp.float32)
            g = jnp.maximum(ga_ref[...] - pb, 0.0)
            h = jnp.maximum(
                jnp.dot(g, w2_ref[...], preferred_element_type=jnp.float32)
                + b2_ref[...], 0.0)
            m = jnp.maximum(
                jnp.dot(h, w3_ref[...], preferred_element_type=jnp.float32)
                + b3_ref[...], 0.0)
            out_ref[...] = m.T

        @pl.when(i >= n_valid_blk)
        def _():
            out_ref[...] = jnp.zeros_like(out_ref)

    return _mlp_body


def _edge_mlp(ga, gp, wp, w2, b2, w3, b3, n_valid_blk):
    e = ga.shape[0]
    f = w3.shape[1]
    nblk = e // _EDGE_BLK
    return pl.pallas_call(
        _make_mlp_body(n_valid_blk),
        grid=(nblk,),
        in_specs=[
            pl.BlockSpec((_EDGE_BLK, 64), lambda i: (i, 0)),
            pl.BlockSpec((_EDGE_BLK, 16), lambda i: (i, 0)),
            pl.BlockSpec((3, 64), lambda i: (0, 0)),
            pl.BlockSpec((64, w2.shape[1]), lambda i: (0, 0)),
            pl.BlockSpec((1, w2.shape[1]), lambda i: (0, 0)),
            pl.BlockSpec((w2.shape[1], f), lambda i: (0, 0)),
            pl.BlockSpec((1, f), lambda i: (0, 0)),
        ],
        out_specs=pl.BlockSpec((f, _EDGE_BLK), lambda i: (0, i)),
        out_shape=jax.ShapeDtypeStruct((f, e), jnp.float32),
    )(ga, gp, wp, w2, b2.reshape(1, -1), w3, b3.reshape(1, -1))


def _post_body(aggp_ref, pos_ref, wg_ref, bg_ref, wx_ref, wp_ref, b1_ref,
               t_ref):
    aggt = jnp.max(aggp_ref[...], axis=0)[:, :pos_ref.shape[0]]
    h = jnp.maximum(
        jax.lax.dot_general(aggt, wg_ref[...], (((0,), (0,)), ((), ())),
                            preferred_element_type=jnp.float32) + bg_ref[...],
        0.0)
    pb = jax.lax.dot_general(pos_ref[...], wp_ref[...], (((1,), (0,)), ((), ())),
                             preferred_element_type=jnp.float32)
    xa = jax.lax.dot_general(h, wx_ref[...], (((1,), (0,)), ((), ())),
                             preferred_element_type=jnp.float32)
    t_ref[...] = xa + pb + b1_ref[...]


def _post_layer1(aggp, pos, wg, bg, wx, wp, b1):
    n = pos.shape[0]
    h2 = wx.shape[1]
    return pl.pallas_call(
        _post_body,
        out_shape=jax.ShapeDtypeStruct((n, h2), jnp.float32),
    )(aggp, pos, wg, bg.reshape(1, -1), wx, wp, b1.reshape(1, -1))


def _head_body(aggp_ref, wg_ref, bg_ref, w1_ref, b1_ref, w2_ref, b2_ref,
               out_ref):
    aggt = jnp.max(aggp_ref[...], axis=0)[:, :out_ref.shape[0]]
    h = jnp.maximum(
        jax.lax.dot_general(aggt, wg_ref[...], (((0,), (0,)), ((), ())),
                            preferred_element_type=jnp.float32) + bg_ref[...],
        0.0)
    z = jnp.maximum(
        jnp.dot(h, w1_ref[...], preferred_element_type=jnp.float32) + b1_ref[...],
        0.0)
    out_ref[...] = (
        jnp.dot(z, w2_ref[...], preferred_element_type=jnp.float32) + b2_ref[...])


def _head(aggp, wg, bg, w1, b1, w2, b2, n):
    return pl.pallas_call(
        _head_body,
        out_shape=jax.ShapeDtypeStruct((n, w2.shape[1]), jnp.float32),
    )(aggp, wg, bg.reshape(1, -1), w1, b1.reshape(1, -1), w2, b2.reshape(1, -1))


def _sc_gather_ap(a_tab, p_tab, src, dst):
    e = src.shape[0]
    epw = e // _NW
    ch = _GATHER_CHUNK
    mesh = plsc.VectorSubcoreMesh(core_axis_name="c", subcore_axis_name="s")

    @functools.partial(
        pl.kernel,
        out_type=(jax.ShapeDtypeStruct((e, 64), jnp.float32),
                  jax.ShapeDtypeStruct((e, 16), jnp.float32)),
        mesh=mesh,
        scratch_types=[
            pltpu.VMEM((ch,), jnp.int32),
            pltpu.VMEM((ch,), jnp.int32),
            pltpu.VMEM((ch, 64), jnp.float32),
            pltpu.VMEM((ch, 16), jnp.float32),
            pltpu.VMEM((ch,), jnp.int32),
            pltpu.VMEM((ch,), jnp.int32),
            pltpu.VMEM((ch, 64), jnp.float32),
            pltpu.VMEM((ch, 16), jnp.float32),
            pltpu.SemaphoreType.DMA,
            pltpu.SemaphoreType.DMA,
            pltpu.SemaphoreType.DMA,
            pltpu.SemaphoreType.DMA,
        ],
        compiler_params=pltpu.CompilerParams(use_tc_tiling_on_sc=False),
    )
    def k(a_hbm, p_hbm, src_hbm, dst_hbm, ga_hbm, gp_hbm, idxs0, idxd0,
          ts0, tp0, idxs1, idxd1, ts1, tp1, sa0, sb0, sa1, sb1):
        wid = lax.axis_index("s") * _NC + lax.axis_index("c")
        base = wid * epw

        @pl.loop(0, epw, step=2 * ch)
        def _(off):
            pltpu.sync_copy(src_hbm.at[pl.ds(base + off, ch)], idxs0)
            pltpu.sync_copy(dst_hbm.at[pl.ds(base + off, ch)], idxd0)
            ca0 = pltpu.async_copy(a_hbm.at[idxs0], ts0, sa0)
            cb0 = pltpu.async_copy(p_hbm.at[idxd0], tp0, sb0)
            pltpu.sync_copy(src_hbm.at[pl.ds(base + off + ch, ch)], idxs1)
            pltpu.sync_copy(dst_hbm.at[pl.ds(base + off + ch, ch)], idxd1)
            ca1 = pltpu.async_copy(a_hbm.at[idxs1], ts1, sa1)
            cb1 = pltpu.async_copy(p_hbm.at[idxd1], tp1, sb1)
            ca0.wait()
            cb0.wait()
            pltpu.sync_copy(ts0, ga_hbm.at[pl.ds(base + off, ch)])
            pltpu.sync_copy(tp0, gp_hbm.at[pl.ds(base + off, ch)])
            ca1.wait()
            cb1.wait()
            pltpu.sync_copy(ts1, ga_hbm.at[pl.ds(base + off + ch, ch)])
            pltpu.sync_copy(tp1, gp_hbm.at[pl.ds(base + off + ch, ch)])

    return k(a_tab, p_tab, src, dst)


def _sc_gather_a(a_tab, src):
    e = src.shape[0]
    epw = e // _NW
    ch = _GATHER_CHUNK
    mesh = plsc.VectorSubcoreMesh(core_axis_name="c", subcore_axis_name="s")

    @functools.partial(
        pl.kernel,
        out_type=jax.ShapeDtypeStruct((e, 64), jnp.float32),
        mesh=mesh,
        scratch_types=[
            pltpu.VMEM((ch,), jnp.int32),
            pltpu.VMEM((ch, 64), jnp.float32),
            pltpu.VMEM((ch,), jnp.int32),
            pltpu.VMEM((ch, 64), jnp.float32),
            pltpu.SemaphoreType.DMA,
            pltpu.SemaphoreType.DMA,
        ],
        compiler_params=pltpu.CompilerParams(use_tc_tiling_on_sc=False),
    )
    def k(a_hbm, src_hbm, ga_hbm, idxs0, ts0, idxs1, ts1, sa0, sa1):
        wid = lax.axis_index("s") * _NC + lax.axis_index("c")
        base = wid * epw

        @pl.loop(0, epw, step=2 * ch)
        def _(off):
            pltpu.sync_copy(src_hbm.at[pl.ds(base + off, ch)], idxs0)
            ca0 = pltpu.async_copy(a_hbm.at[idxs0], ts0, sa0)
            pltpu.sync_copy(src_hbm.at[pl.ds(base + off + ch, ch)], idxs1)
            ca1 = pltpu.async_copy(a_hbm.at[idxs1], ts1, sa1)
            ca0.wait()
            pltpu.sync_copy(ts0, ga_hbm.at[pl.ds(base + off, ch)])
            ca1.wait()
            pltpu.sync_copy(ts1, ga_hbm.at[pl.ds(base + off + ch, ch)])

    return k(a_tab, src)


_FPW = 4


def _sc_segment_max(msg_t, dst, n):
    f, e = msg_t.shape
    n_tr = f // _FPW
    n_seg = _NW // n_tr
    seg = e // n_seg
    ch = _SCAT_CHUNK
    mesh = plsc.VectorSubcoreMesh(core_axis_name="c", subcore_axis_name="s")

    @functools.partial(
        pl.kernel,
        out_type=jax.ShapeDtypeStruct((_NW * _FPW * n,), jnp.float32),
        mesh=mesh,
        scratch_types=[
            pltpu.VMEM((ch,), jnp.int32),
            pltpu.VMEM((_FPW, ch), jnp.float32),
            pltpu.VMEM((ch,), jnp.int32),
            pltpu.VMEM((_FPW, ch), jnp.float32),
        ] + [pltpu.VMEM((n,), jnp.float32) for _ in range(_FPW)] + [
            pltpu.VMEM((n,), jnp.int32),
            pltpu.VMEM((16,), jnp.int32),
            pltpu.SemaphoreType.DMA,
            pltpu.SemaphoreType.DMA,
            pltpu.SemaphoreType.DMA,
            pltpu.SemaphoreType.DMA,
        ],
        compiler_params=pltpu.CompilerParams(needs_layout_passes=False),
    )
    def k(mt_hbm, dst_hbm, agg_hbm, idx0, val0, idx1, val1, a0, a1, a2, a3,
          lanes, mask_v, si0, sv0, si1, sv1):
        accs = (a0, a1, a2, a3)
        wid = lax.axis_index("s") * _NC + lax.axis_index("c")
        tr = wid % n_tr
        sg = wid // n_tr
        base = sg * seg
        zeros16 = jnp.zeros((16,), jnp.float32)
        lane_iota = lax.iota(jnp.int32, 16)

        for fr in range(_FPW):
            @pl.loop(0, n, step=16)
            def _(i):
                accs[fr][pl.ds(i, 16)] = zeros16

        def process(idx_v, val_v):
            @pl.loop(0, ch, step=64)
            def _(i):
                idxs = [idx_v[pl.ds(i + 16 * k2, 16)] for k2 in range(4)]
                for k2 in range(4):
                    plsc.store_scatter(lanes, [idxs[k2]],
                                       lane_iota + 16 * k2)
                rds = [plsc.load_gather(lanes, [idxs[k2]])
                       for k2 in range(4)]
                neq = rds[0] != lane_iota
                for k2 in range(1, 4):
                    neq = jnp.logical_or(neq,
                                         rds[k2] != lane_iota + 16 * k2)
                dup64 = jnp.any(neq)

                @pl.when(jnp.logical_not(dup64))
                def _():
                    curs = [[plsc.load_gather(accs[fr], [idxs[k2]])
                             for fr in range(_FPW)] for k2 in range(4)]
                    for k2 in range(4):
                        for fr in range(_FPW):
                            plsc.store_scatter(
                                accs[fr], [idxs[k2]],
                                jnp.maximum(curs[k2][fr],
                                            val_v[fr, pl.ds(i + 16 * k2,
                                                            16)]))

                @pl.when(dup64)
                def _():
                    for k2 in range(4):
                        idx = idxs[k2]
                        plsc.store_scatter(lanes, [idx], lane_iota)
                        rd = plsc.load_gather(lanes, [idx])
                        vals = [val_v[fr, pl.ds(i + 16 * k2, 16)]
                                for fr in range(_FPW)]
                        has_dup = jnp.any(rd != lane_iota)

                        @pl.when(jnp.logical_not(has_dup))
                        def _():
                            curs2 = [plsc.load_gather(accs[fr], [idx])
                                     for fr in range(_FPW)]
                            for fr in range(_FPW):
                                plsc.store_scatter(
                                    accs[fr], [idx],
                                    jnp.maximum(curs2[fr], vals[fr]))

                        @pl.when(has_dup)
                        def _():
                            for fr in range(_FPW):
                                mask_v[...] = jnp.ones((16,), jnp.int32)

                                @pl.loop(0, 16)
                                def _(t):
                                    f2 = mask_v[...] > 0
                                    cur2 = plsc.load_gather(accs[fr], [idx])
                                    new2 = jnp.maximum(cur2, vals[fr])
                                    plsc.store_scatter(accs[fr], [idx],
                                                       new2, mask=f2)
                                    chk2 = plsc.load_gather(accs[fr], [idx])
                                    mask_v[...] = jnp.where(
                                        f2 & (chk2 < new2), 1,
                                        0).astype(jnp.int32)

        @pl.loop(0, seg, step=2 * ch)
        def _(off):
            ci0 = pltpu.async_copy(dst_hbm.at[pl.ds(base + off, ch)],
                                   idx0, si0)
            cv0 = pltpu.async_copy(
                mt_hbm.at[pl.ds(tr * _FPW, _FPW), pl.ds(base + off, ch)],
                val0, sv0)
            ci1 = pltpu.async_copy(dst_hbm.at[pl.ds(base + off + ch, ch)],
                                   idx1, si1)
            cv1 = pltpu.async_copy(
                mt_hbm.at[pl.ds(tr * _FPW, _FPW),
                          pl.ds(base + off + ch, ch)],
                val1, sv1)
            ci0.wait()
            cv0.wait()
            process(idx0, val0)
            ci1.wait()
            cv1.wait()
            process(idx1, val1)

        for fr in range(_FPW):
            pltpu.async_copy(
                accs[fr],
                agg_hbm.at[pl.ds(((sg * n_tr + tr) * _FPW + fr) * n, n)],
                si0).wait()

    out_flat = k(msg_t, dst)
    return out_flat.reshape(n_seg, f, n)


def kernel(x, pos, edge_index, l1_W1, l1_b1, l1_W2, l1_b2, l1_W3, l1_b3,
           l1_Wg, l1_bg, l2_W1, l2_b1, l2_W2, l2_b2, l2_W3, l2_b3, l2_Wg,
           l2_bg, h_W1, h_b1, h_W2, h_b2):
    n = x.shape[0]
    e = edge_index.shape[1]
    src = edge_index[0].astype(jnp.int32)
    dst = edge_index[1].astype(jnp.int32)
    pos = pos.astype(jnp.float32)
    src_p = jnp.pad(src, (0, _E_PAD - e))
    dst_p = jnp.pad(dst, (0, _E_PAD - e))
    p16 = jnp.pad(pos, ((0, 0), (0, 13)))
    n_valid_blk = e // _EDGE_BLK

    a1 = _make_tables(x, pos, l1_W1[:128], l1_W1[128:], l1_b1)
    ga1, gp = _sc_gather_ap(a1, p16, src_p, dst_p)
    m1t = _edge_mlp(ga1, gp, l1_W1[128:], l1_W2, l1_b2, l1_W3, l1_b3,
                    n_valid_blk)
    agg1p = _sc_segment_max(m1t, dst_p, n)

    a2 = _post_layer1(agg1p, pos, l1_Wg, l1_bg, l2_W1[:32], l2_W1[32:], l2_b1)
    ga2 = _sc_gather_a(a2, src_p)
    m2t = _edge_mlp(ga2, gp, l2_W1[32:], l2_W2, l2_b2, l2_W3, l2_b3,
                    n_valid_blk)
    agg2p = _sc_segment_max(m2t, dst_p, n)

    return _head(agg2p, l2_Wg, l2_bg, h_W1, h_b1, h_W2, h_b2, n)


# speedup vs baseline: 1.0858x; 1.0160x over previous
import functools

import jax
import jax.numpy as jnp
from jax import lax
from jax.experimental import pallas as pl
from jax.experimental.pallas import tpu as pltpu
from jax.experimental.pallas import tpu_sc as plsc

_NC = 2
_NS = 16
_NW = _NC * _NS

_E_PAD = 327680
_GATHER_CHUNK = 512
_SCAT_CHUNK = 4096
_EDGE_BLK = 2560


def _tables_body(xin_ref, pos_ref, wx_ref, wp_ref, b_ref, t_ref):
    pb = jax.lax.dot_general(pos_ref[...], wp_ref[...], (((1,), (0,)), ((), ())),
                             preferred_element_type=jnp.float32)
    xa = jax.lax.dot_general(xin_ref[...], wx_ref[...], (((1,), (0,)), ((), ())),
                             preferred_element_type=jnp.float32)
    t_ref[...] = xa + pb + b_ref[...]


def _make_tables(xin, pos, wx, wp, b):
    n, h = xin.shape[0], wx.shape[1]
    return pl.pallas_call(
        _tables_body,
        out_shape=jax.ShapeDtypeStruct((n, h), jnp.float32),
    )(xin, pos, wx, wp, b.reshape(1, h))


def _make_mlp_body(n_valid_blk):
    def _mlp_body(ga_ref, gp_ref, wp_ref, w2_ref, b2_ref, w3_ref, b3_ref,
                  out_ref):
        i = pl.program_id(0)

        @pl.when(i < n_valid_blk)
        def _():
            pb = jnp.dot(gp_ref[:, :3], wp_ref[...],
                         preferred_element_type=jnp.float32)
            g = jnp.maximum(ga_ref[...] - pb, 0.0)
            h = jnp.maximum(
                jnp.dot(g, w2_ref[...], preferred_element_type=jnp.float32)
                + b2_ref[...], 0.0)
            m = jnp.maximum(
                jnp.dot(h, w3_ref[...], preferred_element_type=jnp.float32)
                + b3_ref[...], 0.0)
            out_ref[...] = m.T

        @pl.when(i >= n_valid_blk)
        def _():
            out_ref[...] = jnp.zeros_like(out_ref)

    return _mlp_body


def _edge_mlp(ga, gp, wp, w2, b2, w3, b3, n_valid_blk):
    e = ga.shape[0]
    f = w3.shape[1]
    nblk = e // _EDGE_BLK
    return pl.pallas_call(
        _make_mlp_body(n_valid_blk),
        grid=(nblk,),
        in_specs=[
            pl.BlockSpec((_EDGE_BLK, 64), lambda i: (i, 0)),
            pl.BlockSpec((_EDGE_BLK, 16), lambda i: (i, 0)),
            pl.BlockSpec((3, 64), lambda i: (0, 0)),
            pl.BlockSpec((64, w2.shape[1]), lambda i: (0, 0)),
            pl.BlockSpec((1, w2.shape[1]), lambda i: (0, 0)),
            pl.BlockSpec((w2.shape[1], f), lambda i: (0, 0)),
            pl.BlockSpec((1, f), lambda i: (0, 0)),
        ],
        out_specs=pl.BlockSpec((f, _EDGE_BLK), lambda i: (0, i)),
        out_shape=jax.ShapeDtypeStruct((f, e), jnp.float32),
    )(ga, gp, wp, w2, b2.reshape(1, -1), w3, b3.reshape(1, -1))


def _post_body(aggp_ref, pos_ref, wg_ref, bg_ref, wx_ref, wp_ref, b1_ref,
               t_ref):
    aggt = jnp.max(aggp_ref[...], axis=0)[:, :pos_ref.shape[0]]
    h = jnp.maximum(
        jax.lax.dot_general(aggt, wg_ref[...], (((0,), (0,)), ((), ())),
                            preferred_element_type=jnp.float32) + bg_ref[...],
        0.0)
    pb = jax.lax.dot_general(pos_ref[...], wp_ref[...], (((1,), (0,)), ((), ())),
                             preferred_element_type=jnp.float32)
    xa = jax.lax.dot_general(h, wx_ref[...], (((1,), (0,)), ((), ())),
                             preferred_element_type=jnp.float32)
    t_ref[...] = xa + pb + b1_ref[...]


def _post_layer1(aggp, pos, wg, bg, wx, wp, b1):
    n = pos.shape[0]
    h2 = wx.shape[1]
    return pl.pallas_call(
        _post_body,
        out_shape=jax.ShapeDtypeStruct((n, h2), jnp.float32),
    )(aggp, pos, wg, bg.reshape(1, -1), wx, wp, b1.reshape(1, -1))


def _head_body(aggp_ref, wg_ref, bg_ref, w1_ref, b1_ref, w2_ref, b2_ref,
               out_ref):
    aggt = jnp.max(aggp_ref[...], axis=0)[:, :out_ref.shape[0]]
    h = jnp.maximum(
        jax.lax.dot_general(aggt, wg_ref[...], (((0,), (0,)), ((), ())),
                            preferred_element_type=jnp.float32) + bg_ref[...],
        0.0)
    z = jnp.maximum(
        jnp.dot(h, w1_ref[...], preferred_element_type=jnp.float32) + b1_ref[...],
        0.0)
    out_ref[...] = (
        jnp.dot(z, w2_ref[...], preferred_element_type=jnp.float32) + b2_ref[...])


def _head(aggp, wg, bg, w1, b1, w2, b2, n):
    return pl.pallas_call(
        _head_body,
        out_shape=jax.ShapeDtypeStruct((n, w2.shape[1]), jnp.float32),
    )(aggp, wg, bg.reshape(1, -1), w1, b1.reshape(1, -1), w2, b2.reshape(1, -1))


def _sc_gather_ap(a_tab, p_tab, src, dst):
    e = src.shape[0]
    epw = e // _NW
    ch = _GATHER_CHUNK
    mesh = plsc.VectorSubcoreMesh(core_axis_name="c", subcore_axis_name="s")

    @functools.partial(
        pl.kernel,
        out_type=(jax.ShapeDtypeStruct((e, 64), jnp.float32),
                  jax.ShapeDtypeStruct((e, 16), jnp.float32)),
        mesh=mesh,
        scratch_types=[
            pltpu.VMEM((ch,), jnp.int32),
            pltpu.VMEM((ch,), jnp.int32),
            pltpu.VMEM((ch, 64), jnp.float32),
            pltpu.VMEM((ch, 16), jnp.float32),
            pltpu.VMEM((ch,), jnp.int32),
            pltpu.VMEM((ch,), jnp.int32),
            pltpu.VMEM((ch, 64), jnp.float32),
            pltpu.VMEM((ch, 16), jnp.float32),
            pltpu.SemaphoreType.DMA,
            pltpu.SemaphoreType.DMA,
            pltpu.SemaphoreType.DMA,
            pltpu.SemaphoreType.DMA,
        ],
        compiler_params=pltpu.CompilerParams(use_tc_tiling_on_sc=False),
    )
    def k(a_hbm, p_hbm, src_hbm, dst_hbm, ga_hbm, gp_hbm, idxs0, idxd0,
          ts0, tp0, idxs1, idxd1, ts1, tp1, sa0, sb0, sa1, sb1):
        wid = lax.axis_index("s") * _NC + lax.axis_index("c")
        base = wid * epw

        @pl.loop(0, epw, step=2 * ch)
        def _(off):
            pltpu.sync_copy(src_hbm.at[pl.ds(base + off, ch)], idxs0)
            pltpu.sync_copy(dst_hbm.at[pl.ds(base + off, ch)], idxd0)
            ca0 = pltpu.async_copy(a_hbm.at[idxs0], ts0, sa0)
            cb0 = pltpu.async_copy(p_hbm.at[idxd0], tp0, sb0)
            pltpu.sync_copy(src_hbm.at[pl.ds(base + off + ch, ch)], idxs1)
            pltpu.sync_copy(dst_hbm.at[pl.ds(base + off + ch, ch)], idxd1)
            ca1 = pltpu.async_copy(a_hbm.at[idxs1], ts1, sa1)
            cb1 = pltpu.async_copy(p_hbm.at[idxd1], tp1, sb1)
            ca0.wait()
            cb0.wait()
            pltpu.sync_copy(ts0, ga_hbm.at[pl.ds(base + off, ch)])
            pltpu.sync_copy(tp0, gp_hbm.at[pl.ds(base + off, ch)])
            ca1.wait()
            cb1.wait()
            pltpu.sync_copy(ts1, ga_hbm.at[pl.ds(base + off + ch, ch)])
            pltpu.sync_copy(tp1, gp_hbm.at[pl.ds(base + off + ch, ch)])

    return k(a_tab, p_tab, src, dst)


def _sc_gather_a(a_tab, src):
    e = src.shape[0]
    epw = e // _NW
    ch = _GATHER_CHUNK
    mesh = plsc.VectorSubcoreMesh(core_axis_name="c", subcore_axis_name="s")

    @functools.partial(
        pl.kernel,
        out_type=jax.ShapeDtypeStruct((e, 64), jnp.float32),
        mesh=mesh,
        scratch_types=[
            pltpu.VMEM((ch,), jnp.int32),
            pltpu.VMEM((ch, 64), jnp.float32),
            pltpu.VMEM((ch,), jnp.int32),
            pltpu.VMEM((ch, 64), jnp.float32),
            pltpu.SemaphoreType.DMA,
            pltpu.SemaphoreType.DMA,
        ],
        compiler_params=pltpu.CompilerParams(use_tc_tiling_on_sc=False),
    )
    def k(a_hbm, src_hbm, ga_hbm, idxs0, ts0, idxs1, ts1, sa0, sa1):
        wid = lax.axis_index("s") * _NC + lax.axis_index("c")
        base = wid * epw

        @pl.loop(0, epw, step=2 * ch)
        def _(off):
            pltpu.sync_copy(src_hbm.at[pl.ds(base + off, ch)], idxs0)
            ca0 = pltpu.async_copy(a_hbm.at[idxs0], ts0, sa0)
            pltpu.sync_copy(src_hbm.at[pl.ds(base + off + ch, ch)], idxs1)
            ca1 = pltpu.async_copy(a_hbm.at[idxs1], ts1, sa1)
            ca0.wait()
            pltpu.sync_copy(ts0, ga_hbm.at[pl.ds(base + off, ch)])
            ca1.wait()
            pltpu.sync_copy(ts1, ga_hbm.at[pl.ds(base + off + ch, ch)])

    return k(a_tab, src)


_FPW = 4


def _sc_segment_max(msg_t, dst, n):
    f, e = msg_t.shape
    n_tr = f // _FPW
    n_seg = _NW // n_tr
    seg = e // n_seg
    ch = _SCAT_CHUNK
    mesh = plsc.VectorSubcoreMesh(core_axis_name="c", subcore_axis_name="s")

    @functools.partial(
        pl.kernel,
        out_type=jax.ShapeDtypeStruct((_NW * _FPW * n,), jnp.float32),
        mesh=mesh,
        scratch_types=[
            pltpu.VMEM((ch,), jnp.int32),
            pltpu.VMEM((_FPW, ch), jnp.float32),
            pltpu.VMEM((ch,), jnp.int32),
            pltpu.VMEM((_FPW, ch), jnp.float32),
        ] + [pltpu.VMEM((n,), jnp.float32) for _ in range(_FPW)] + [
            pltpu.VMEM((n,), jnp.int32),
            pltpu.VMEM((16,), jnp.int32),
            pltpu.SemaphoreType.DMA,
            pltpu.SemaphoreType.DMA,
            pltpu.SemaphoreType.DMA,
            pltpu.SemaphoreType.DMA,
        ],
        compiler_params=pltpu.CompilerParams(needs_layout_passes=False),
    )
    def k(mt_hbm, dst_hbm, agg_hbm, idx0, val0, idx1, val1, a0, a1, a2, a3,
          lanes, mask_v, si0, sv0, si1, sv1):
        accs = (a0, a1, a2, a3)
        wid = lax.axis_index("s") * _NC + lax.axis_index("c")
        tr = wid % n_tr
        sg = wid // n_tr
        base = sg * seg
        zeros16 = jnp.zeros((16,), jnp.float32)
        lane_iota = lax.iota(jnp.int32, 16)

        for fr in range(_FPW):
            @pl.loop(0, n, step=16)
            def _(i):
                accs[fr][pl.ds(i, 16)] = zeros16

        def process(idx_v, val_v):
            @pl.loop(0, ch, step=64)
            def _(i):
                idxs = [idx_v[pl.ds(i + 16 * k2, 16)] for k2 in range(4)]
                for k2 in range(4):
                    plsc.store_scatter(lanes, [idxs[k2]],
                                       lane_iota + 16 * k2)
                rds = [plsc.load_gather(lanes, [idxs[k2]])
                       for k2 in range(4)]
                neq = rds[0] != lane_iota
                for k2 in range(1, 4):
                    neq = jnp.logical_or(neq,
                                         rds[k2] != lane_iota + 16 * k2)
                dup64 = jnp.any(neq)

                @pl.when(jnp.logical_not(dup64))
                def _():
                    curs = [[plsc.load_gather(accs[fr], [idxs[k2]])
                             for fr in range(_FPW)] for k2 in range(4)]
                    for k2 in range(4):
                        for fr in range(_FPW):
                            plsc.store_scatter(
                                accs[fr], [idxs[k2]],
                                jnp.maximum(curs[k2][fr],
                                            val_v[fr, pl.ds(i + 16 * k2,
                                                            16)]))

                @pl.when(dup64)
                def _():
                    for k2 in range(4):
                        idx = idxs[k2]
                        plsc.store_scatter(lanes, [idx], lane_iota)
                        rd = plsc.load_gather(lanes, [idx])
                        vals = [val_v[fr, pl.ds(i + 16 * k2, 16)]
                                for fr in range(_FPW)]
                        has_dup = jnp.any(rd != lane_iota)

                        @pl.when(jnp.logical_not(has_dup))
                        def _():
                            curs2 = [plsc.load_gather(accs[fr], [idx])
                                     for fr in range(_FPW)]
                            for fr in range(_FPW):
                                plsc.store_scatter(
                                    accs[fr], [idx],
                                    jnp.maximum(curs2[fr], vals[fr]))

                        @pl.when(has_dup)
                        def _():
                            for fr in range(_FPW):
                                mask_v[...] = jnp.ones((16,), jnp.int32)

                                @pl.loop(0, 16)
                                def _(t):
                                    f2 = mask_v[...] > 0
                                    cur2 = plsc.load_gather(accs[fr], [idx])
                                    new2 = jnp.maximum(cur2, vals[fr])
                                    plsc.store_scatter(accs[fr], [idx],
                                                       new2, mask=f2)
                                    chk2 = plsc.load_gather(accs[fr], [idx])
                                    mask_v[...] = jnp.where(
                                        f2 & (chk2 < new2), 1,
                                        0).astype(jnp.int32)

        @pl.loop(0, seg, step=2 * ch)
        def _(off):
            ci0 = pltpu.async_copy(dst_hbm.at[pl.ds(base + off, ch)],
                                   idx0, si0)
            cv0 = pltpu.async_copy(
                mt_hbm.at[pl.ds(tr * _FPW, _FPW), pl.ds(base + off, ch)],
                val0, sv0)
            ci1 = pltpu.async_copy(dst_hbm.at[pl.ds(base + off + ch, ch)],
                                   idx1, si1)
            cv1 = pltpu.async_copy(
                mt_hbm.at[pl.ds(tr * _FPW, _FPW),
                          pl.ds(base + off + ch, ch)],
                val1, sv1)
            ci0.wait()
            cv0.wait()
            process(idx0, val0)
            ci1.wait()
            cv1.wait()
            process(idx1, val1)

        for fr in range(_FPW):
            pltpu.async_copy(
                accs[fr],
                agg_hbm.at[pl.ds(((sg * n_tr + tr) * _FPW + fr) * n, n)],
                si0).wait()

    out_flat = k(msg_t, dst)
    return out_flat.reshape(n_seg, f, n)


def kernel(x, pos, edge_index, l1_W1, l1_b1, l1_W2, l1_b2, l1_W3, l1_b3,
           l1_Wg, l1_bg, l2_W1, l2_b1, l2_W2, l2_b2, l2_W3, l2_b3, l2_Wg,
           l2_bg, h_W1, h_b1, h_W2, h_b2):
    n = x.shape[0]
    e = edge_index.shape[1]
    src = edge_index[0].astype(jnp.int32)
    dst = edge_index[1].astype(jnp.int32)
    pos = pos.astype(jnp.float32)
    src_p = jnp.pad(src, (0, _E_PAD - e))
    dst_p = jnp.pad(dst, (0, _E_PAD - e))
    p16 = jnp.pad(pos, ((0, 0), (0, 13)))
    n_valid_blk = e // _EDGE_BLK

    a1 = _make_tables(x, pos, l1_W1[:128], l1_W1[128:], l1_b1)
    ga1, gp = _sc_gather_ap(a1, p16, src_p, dst_p)
    m1t = _edge_mlp(ga1, gp, l1_W1[128:], l1_W2, l1_b2, l1_W3, l1_b3,
                    n_valid_blk)
    agg1p = _sc_segment_max(m1t, dst_p, n)

    a2 = _post_layer1(agg1p, pos, l1_Wg, l1_bg, l2_W1[:32], l2_W1[32:], l2_b1)
    ga2 = _sc_gather_a(a2, src_p)
    m2t = _edge_mlp(ga2, gp, l2_W1[32:], l2_W2, l2_b2, l2_W3, l2_b3,
                    n_valid_blk)
    agg2p = _sc_segment_max(m2t, dst_p, n)

    return _head(agg2p, l2_Wg, l2_bg, h_W1, h_b1, h_W2, h_b2, n)
